# baseline (speedup 1.0000x reference)
import functools

import jax
import jax.numpy as jnp
from jax import lax
from jax.experimental import pallas as pl
from jax.experimental.pallas import tpu as pltpu
from jax.experimental.pallas import tpu_sc as plsc

D = 8192
K = 32
ROWS_PER_BLOCK = 128
NEG = -3.38e38
MASK31 = 0x7FFFFFFF

_NUM_WORKERS = 32
_ROWS_PER_WORKER = D // _NUM_WORKERS


def _f32_to_key(v):
    b = plsc.bitcast(v, jnp.int32)
    return b ^ ((b >> 31) & MASK31)


def _key_to_f32(k):
    return plsc.bitcast(k ^ ((k >> 31) & MASK31), jnp.float32)


def _process_row(w_hbm, rowref, semref, candbuf, keybuf, repbuf, base_row, rloc):
    pltpu.make_async_copy(w_hbm.at[base_row + rloc], rowref, semref).wait()

    g = base_row + rloc
    off = (g // 16) * 16
    lane = g - off
    dv = rowref[pl.ds(off, 16)]
    rowref[pl.ds(off, 16)] = jnp.where(
        lax.iota(jnp.int32, 16) == lane, jnp.float32(0.0), dv)

    def s1(o, c):
        m1, m2 = c
        g = rowref[pl.ds(o * 128, 16)]
        for u in range(1, 8):
            g = jnp.maximum(g, rowref[pl.ds(o * 128 + u * 16, 16)])
        m2 = jnp.maximum(m2, jnp.minimum(m1, g))
        m1 = jnp.maximum(m1, g)
        return (m1, m2)

    m1, m2 = lax.fori_loop(
        0, 64, s1,
        (jnp.full((16,), NEG, jnp.float32), jnp.full((16,), NEG, jnp.float32)))
    tau = -jnp.max(-m2)
    rmax = jnp.max(m1)
    tau_v = jnp.full((16,), tau)

    def s2(i, cnt_v):
        for u in range(4):
            vv = rowref[pl.ds(i * 64 + u * 16, 16)]
            msk = vv >= tau_v
            inc = jnp.where(msk, jnp.int32(1), jnp.int32(0))
            pos = cnt_v + plsc.cumsum(inc) - 1
            plsc.store_scatter(candbuf, [pos], vv, mask=msk)
            cnt_v = cnt_v + plsc.all_reduce_population_count(msk)
        return cnt_v

    cnt_v = lax.fori_loop(0, 128, s2, jnp.zeros((16,), jnp.int32))
    cnt = cnt_v[0]

    candbuf[pl.ds(cnt, 16)] = jnp.full((16,), NEG, jnp.float32)
    nv = (cnt + 15) // 16

    def kt(i, c):
        keybuf[pl.ds(i * 16, 16)] = _f32_to_key(candbuf[pl.ds(i * 16, 16)])
        return c

    lax.fori_loop(0, nv, kt, jnp.int32(0))

    def count_ge(cand):
        cs = jnp.full((16,), cand)

        def cb(i, a):
            kk = keybuf[pl.ds(i * 16, 16)]
            return a + jnp.where(kk >= cs, jnp.int32(1), jnp.int32(0))

        return jnp.sum(lax.fori_loop(0, nv, cb, jnp.zeros((16,), jnp.int32)))

    lo0 = jnp.max(_f32_to_key(tau_v))
    hi0 = jnp.max(_f32_to_key(jnp.full((16,), rmax))) + 1

    def wcond(c):
        lo, hi = c
        return hi - lo > 1

    def wbody(c):
        lo, hi = c
        mid = lo + ((hi - lo) >> 1)
        n = count_ge(mid)
        return (jnp.where(n >= K, mid, lo), jnp.where(n >= K, hi, mid))

    lo, _ = lax.while_loop(wcond, wbody, (lo0, hi0))

    fv = _key_to_f32(jnp.full((16,), lo))
    rb = (rloc % 16) * 128
    for k2 in range(8):
        repbuf[pl.ds(rb + k2 * 16, 16)] = fv

    nxt = jnp.minimum(rloc + 2, _ROWS_PER_WORKER - 1)
    pltpu.make_async_copy(w_hbm.at[base_row + nxt], rowref, semref).start()


@functools.partial(
    pl.kernel,
    out_type=jax.ShapeDtypeStruct((D * 128,), jnp.float32),
    mesh=plsc.VectorSubcoreMesh(core_axis_name="c", subcore_axis_name="s"),
    compiler_params=pltpu.CompilerParams(needs_layout_passes=False),
    scratch_types=[
        pltpu.VMEM((D,), jnp.float32),
        pltpu.VMEM((D,), jnp.float32),
        pltpu.VMEM((D + 16,), jnp.float32),
        pltpu.VMEM((D + 16,), jnp.int32),
        pltpu.VMEM((2048,), jnp.float32),
        pltpu.SemaphoreType.DMA,
        pltpu.SemaphoreType.DMA,
    ],
)
def _sc_thresh(w_hbm, out_hbm, rowbuf0, rowbuf1, candbuf, keybuf, repbuf,
               sem0, sem1):
    wid = lax.axis_index("s") * 2 + lax.axis_index("c")
    base_row = wid * _ROWS_PER_WORKER

    pltpu.make_async_copy(w_hbm.at[base_row], rowbuf0, sem0).start()
    pltpu.make_async_copy(w_hbm.at[base_row + 1], rowbuf1, sem1).start()

    def body(p, c):
        _process_row(w_hbm, rowbuf0, sem0, candbuf, keybuf, repbuf,
                     base_row, 2 * p)
        _process_row(w_hbm, rowbuf1, sem1, candbuf, keybuf, repbuf,
                     base_row, 2 * p + 1)

        @pl.when((p % 8) == 7)
        def _():
            fl = base_row + 16 * (p // 8)
            pltpu.sync_copy(repbuf, out_hbm.at[pl.ds(fl * 128, 2048)])

        return c

    lax.fori_loop(0, _ROWS_PER_WORKER // 2, body, jnp.int32(0))

    last = base_row + _ROWS_PER_WORKER - 1
    pltpu.make_async_copy(w_hbm.at[last], rowbuf0, sem0).wait()
    pltpu.make_async_copy(w_hbm.at[last], rowbuf1, sem1).wait()


def _soft_kernel(w_ref, soft_ref):
    pid = pl.program_id(0)
    w = w_ref[...]
    r, d = w.shape
    row_ids = pid * r + lax.broadcasted_iota(jnp.int32, (r, d), 0)
    col_ids = lax.broadcasted_iota(jnp.int32, (r, d), 1)
    w = jnp.where(col_ids == row_ids, jnp.float32(0.0), w)
    soft_ref[...] = 1.0 / (1.0 + jnp.exp(w * -5.0))


def _hard_kernel(w_ref, t_ref, hard_ref):
    pid = pl.program_id(0)
    w = w_ref[...]
    r, d = w.shape
    row_ids = pid * r + lax.broadcasted_iota(jnp.int32, (r, d), 0)
    col_ids = lax.broadcasted_iota(jnp.int32, (r, d), 1)
    w = jnp.where(col_ids == row_ids, jnp.float32(0.0), w)
    t = t_ref[:, 0:1]
    hard = (w >= t) & (w > 0.0)
    hard_ref[...] = hard.astype(jnp.float32)


@jax.jit
def kernel(W_raw):
    grid = (D // ROWS_PER_BLOCK,)
    soft = pl.pallas_call(
        _soft_kernel,
        grid=grid,
        in_specs=[pl.BlockSpec((ROWS_PER_BLOCK, D), lambda i: (i, 0))],
        out_specs=pl.BlockSpec((ROWS_PER_BLOCK, D), lambda i: (i, 0)),
        out_shape=jax.ShapeDtypeStruct((D, D), jnp.float32),
    )(W_raw)

    thresh_rep = _sc_thresh(W_raw).reshape(D, 128)

    hard = pl.pallas_call(
        _hard_kernel,
        grid=grid,
        in_specs=[
            pl.BlockSpec((ROWS_PER_BLOCK, D), lambda i: (i, 0)),
            pl.BlockSpec((ROWS_PER_BLOCK, 128), lambda i: (i, 0)),
        ],
        out_specs=pl.BlockSpec((ROWS_PER_BLOCK, D), lambda i: (i, 0)),
        out_shape=jax.ShapeDtypeStruct((D, D), jnp.float32),
    )(W_raw, thresh_rep)

    return (soft, hard)


# speedup vs baseline: 1.9940x; 1.9940x over previous
import jax
import jax.numpy as jnp
from jax.experimental import pallas as pl

D = 8192
K = 32
ROWS_PER_BLOCK = 128
MASK31 = 0x7FFFFFFF


def _key_to_f32(k):
    return jax.lax.bitcast_convert_type(k ^ ((k >> 31) & MASK31), jnp.float32)


def _kernel(w_ref, soft_ref, hard_ref):
    pid = pl.program_id(0)
    w = w_ref[...]
    r, d = w.shape

    row_ids = pid * r + jax.lax.broadcasted_iota(jnp.int32, (r, d), 0)
    col_ids = jax.lax.broadcasted_iota(jnp.int32, (r, d), 1)
    w = jnp.where(col_ids == row_ids, jnp.float32(0.0), w)

    soft_ref[...] = 1.0 / (1.0 + jnp.exp(w * -5.0))

    lo = jnp.full((r, 1), jnp.int32(-2147483648))
    cnt0 = jnp.sum((w >= 0.0).astype(jnp.int32), axis=1, keepdims=True)
    lo = jnp.where(cnt0 >= K, jnp.int32(0), lo)
    for j in range(30, -1, -1):
        cand = lo | jnp.int32(1 << j)
        fcand = _key_to_f32(cand)
        cnt = jnp.sum((w >= fcand).astype(jnp.int32), axis=1, keepdims=True)
        lo = jnp.where(cnt >= K, cand, lo)

    flo = _key_to_f32(lo)
    hard = (w >= flo) & (w > 0.0)
    hard_ref[...] = hard.astype(jnp.float32)


@jax.jit
def kernel(W_raw):
    grid = (D // ROWS_PER_BLOCK,)
    soft, hard = pl.pallas_call(
        _kernel,
        grid=grid,
        in_specs=[pl.BlockSpec((ROWS_PER_BLOCK, D), lambda i: (i, 0))],
        out_specs=[
            pl.BlockSpec((ROWS_PER_BLOCK, D), lambda i: (i, 0)),
            pl.BlockSpec((ROWS_PER_BLOCK, D), lambda i: (i, 0)),
        ],
        out_shape=[
            jax.ShapeDtypeStruct((D, D), jnp.float32),
            jax.ShapeDtypeStruct((D, D), jnp.float32),
        ],
    )(W_raw)
    return (soft, hard)


# speedup vs baseline: 2.0182x; 1.0122x over previous
import jax
import jax.numpy as jnp
from jax.experimental import pallas as pl

D = 8192
K = 32
ROWS_PER_BLOCK = 128


def _kernel(w_ref, soft_ref, hard_ref):
    pid = pl.program_id(0)
    w = w_ref[...]
    r, d = w.shape

    row_ids = pid * r + jax.lax.broadcasted_iota(jnp.int32, (r, d), 0)
    col_ids = jax.lax.broadcasted_iota(jnp.int32, (r, d), 1)
    w = jnp.where(col_ids == row_ids, jnp.float32(0.0), w)

    soft_ref[...] = 1.0 / (1.0 + jnp.exp(w * -5.0))

    b = jax.lax.bitcast_convert_type(w, jnp.int32)
    key = b ^ ((b >> 31) & jnp.int32(0x7FFFFFFF))

    lo = jnp.full((r, 1), jnp.int32(-2147483648))
    cnt0 = jnp.sum((key >= 0).astype(jnp.int32), axis=1, keepdims=True)
    lo = jnp.where(cnt0 >= K, jnp.int32(0), lo)
    for j in range(30, -1, -1):
        cand = lo | jnp.int32(1 << j)
        cnt = jnp.sum((key >= cand).astype(jnp.int32), axis=1, keepdims=True)
        lo = jnp.where(cnt >= K, cand, lo)

    hard = (key >= lo) & (w > 0.0)
    hard_ref[...] = hard.astype(jnp.float32)


@jax.jit
def kernel(W_raw):
    grid = (D // ROWS_PER_BLOCK,)
    soft, hard = pl.pallas_call(
        _kernel,
        grid=grid,
        in_specs=[pl.BlockSpec((ROWS_PER_BLOCK, D), lambda i: (i, 0))],
        out_specs=[
            pl.BlockSpec((ROWS_PER_BLOCK, D), lambda i: (i, 0)),
            pl.BlockSpec((ROWS_PER_BLOCK, D), lambda i: (i, 0)),
        ],
        out_shape=[
            jax.ShapeDtypeStruct((D, D), jnp.float32),
            jax.ShapeDtypeStruct((D, D), jnp.float32),
        ],
    )(W_raw)
    return (soft, hard)
